# Initial kernel scaffold; baseline (speedup 1.0000x reference)
import jax
import jax.numpy as jnp
from jax.experimental import pallas as pl


def kernel(input_ids, weight):
    raise NotImplementedError("write your pallas kernel here")


# speedup vs baseline: 1.6858x; 1.6858x over previous
import functools

import jax
import jax.numpy as jnp
from jax import lax
from jax.experimental import pallas as pl
from jax.experimental.pallas import tpu as pltpu
from jax.experimental.pallas import tpu_sc as plsc

NUM_ROWS = 16384 * 50
DIM = 64
NC = 2
NS = 16
NW = NC * NS
PER_W = NUM_ROWS // NW
CHUNK = 128
N_CHUNKS = PER_W // CHUNK

_mesh = plsc.VectorSubcoreMesh(core_axis_name="c", subcore_axis_name="s")


@functools.partial(
    pl.kernel,
    mesh=_mesh,
    out_type=jax.ShapeDtypeStruct((NUM_ROWS, DIM), jnp.float32),
    scratch_types=[
        pltpu.VMEM((N_CHUNKS, CHUNK), jnp.int32),
        pltpu.VMEM((CHUNK, DIM), jnp.float32),
        pltpu.SemaphoreType.DMA,
    ],
    compiler_params=pltpu.CompilerParams(use_tc_tiling_on_sc=False),
)
def _gather_kernel(idx_hbm, table_hbm, out_hbm, idx_v, rows_v, sem):
    wid = lax.axis_index("s") * NC + lax.axis_index("c")
    chunk0 = pl.multiple_of(wid * N_CHUNKS, 8)
    pltpu.sync_copy(idx_hbm.at[pl.ds(chunk0, N_CHUNKS)], idx_v)

    def body(j, carry):
        pltpu.async_copy(table_hbm.at[idx_v.at[j]], rows_v, sem).wait()
        row0 = pl.multiple_of((chunk0 + j) * CHUNK, 8)
        pltpu.sync_copy(rows_v, out_hbm.at[pl.ds(row0, CHUNK)])
        return carry

    lax.fori_loop(0, N_CHUNKS, body, 0)


def kernel(input_ids, weight):
    idx = input_ids.reshape(NUM_ROWS // CHUNK, CHUNK).astype(jnp.int32)
    out = _gather_kernel(idx, weight)
    return out.reshape(16384, 50, DIM)


# speedup vs baseline: 1.8745x; 1.1120x over previous
import functools

import jax
import jax.numpy as jnp
from jax import lax
from jax.experimental import pallas as pl
from jax.experimental.pallas import tpu as pltpu
from jax.experimental.pallas import tpu_sc as plsc

NUM_ROWS = 16384 * 50
DIM = 64
NC = 2
NS = 16
NW = NC * NS
PER_W = NUM_ROWS // NW
CHUNK = 128
N_CHUNKS = PER_W // CHUNK
NBUF = 8
N_GROUPS = N_CHUNKS // NBUF

_mesh = plsc.VectorSubcoreMesh(core_axis_name="c", subcore_axis_name="s")


@functools.partial(
    pl.kernel,
    mesh=_mesh,
    out_type=jax.ShapeDtypeStruct((NUM_ROWS, DIM), jnp.float32),
    scratch_types=[
        pltpu.VMEM((N_CHUNKS, CHUNK), jnp.int32),
        pltpu.VMEM((NBUF, CHUNK, DIM), jnp.float32),
        pltpu.SemaphoreType.DMA((NBUF,)),
        pltpu.SemaphoreType.DMA((NBUF,)),
    ],
    compiler_params=pltpu.CompilerParams(use_tc_tiling_on_sc=False),
)
def _gather_kernel(idx_hbm, table_hbm, out_hbm, idx_v, rows_v, sem_g, sem_w):
    wid = lax.axis_index("s") * NC + lax.axis_index("c")
    chunk0 = pl.multiple_of(wid * N_CHUNKS, 8)
    pltpu.sync_copy(idx_hbm.at[pl.ds(chunk0, N_CHUNKS)], idx_v)

    def start_gather(b, j):
        pltpu.make_async_copy(
            table_hbm.at[idx_v.at[j]], rows_v.at[b], sem_g.at[b]
        ).start()

    def wait_gather(b):
        pltpu.make_async_copy(
            table_hbm.at[idx_v.at[0]], rows_v.at[b], sem_g.at[b]
        ).wait()

    def start_write(b, j):
        row0 = pl.multiple_of((chunk0 + j) * CHUNK, 8)
        pltpu.make_async_copy(
            rows_v.at[b], out_hbm.at[pl.ds(row0, CHUNK)], sem_w.at[b]
        ).start()

    def wait_write(b, j):
        row0 = pl.multiple_of((chunk0 + j) * CHUNK, 8)
        pltpu.make_async_copy(
            rows_v.at[b], out_hbm.at[pl.ds(row0, CHUNK)], sem_w.at[b]
        ).wait()

    for b in range(NBUF):
        start_gather(b, b)

    def body(g, carry):
        for b in range(NBUF):
            j = g * NBUF + b
            wait_gather(b)
            start_write(b, j)
        for b in range(NBUF):
            j = g * NBUF + b
            wait_write(b, j)
            start_gather(b, j + NBUF)
        return carry

    lax.fori_loop(0, N_GROUPS - 1, body, 0)

    g_last = N_GROUPS - 1
    for b in range(NBUF):
        j = g_last * NBUF + b
        wait_gather(b)
        start_write(b, j)
    for b in range(NBUF):
        j = g_last * NBUF + b
        wait_write(b, j)


def kernel(input_ids, weight):
    idx = input_ids.reshape(NUM_ROWS // CHUNK, CHUNK).astype(jnp.int32)
    out = _gather_kernel(idx, weight)
    return out.reshape(16384, 50, DIM)
